# baseline (speedup 1.0000x reference)
import functools

import jax
import jax.numpy as jnp
from jax import lax
from jax.experimental import pallas as pl
from jax.experimental.pallas import tpu as pltpu
from jax.experimental.pallas import tpu_sc as plsc

_N = 10000
_E = 320000
_F = 128
_H = 32
_C = 10
_G = 64

_NC, _NS = 2, 16
_NW = _NC * _NS
_NPAD = 10240
_B = 128
_RG = 8
_EPAD = _NW * 80 * _B
_IDXROWS = _EPAD // _B
_ROWS_PER_W = _IDXROWS // _NW
_NGRP = _ROWS_PER_W // _RG
_RPS = _NPAD // _NS
_RGA = 4
_NGRPA = _ROWS_PER_W // _RGA
_RA = 72
_RB = 160 - _RA

_sc_mesh = plsc.VectorSubcoreMesh(core_axis_name="c", subcore_axis_name="s")


@functools.partial(
    pl.kernel,
    out_type=jax.ShapeDtypeStruct((_NC * _NPAD,), jnp.float32),
    mesh=_sc_mesh,
    scratch_types=[
        pltpu.VMEM((_ROWS_PER_W, _B), jnp.int32),
        pltpu.VMEM((_B,), jnp.float32),
        pltpu.VMEM((_RPS,), jnp.float32),
        pltpu.VMEM_SHARED((_NPAD,), jnp.float32),
        pltpu.SemaphoreType.DMA,
    ],
)
def _sc_degree(dst_hbm, out_hbm, didx_v, ones_v, zrow_v, acc_sh, sem):
    cid = lax.axis_index("c")
    sid = lax.axis_index("s")
    wid = sid * _NC + cid
    row0 = wid * _ROWS_PER_W
    idesc = [
        pltpu.async_copy(dst_hbm.at[pl.ds(row0 + g * _RG, _RG)],
                         didx_v.at[pl.ds(g * _RG, _RG)], sem)
        for g in range(_NGRP)
    ]
    for i in range(_B // 16):
        ones_v[pl.ds(i * 16, 16)] = jnp.ones((16,), jnp.float32)
    for i in range(_RPS // 16):
        zrow_v[pl.ds(i * 16, 16)] = jnp.zeros((16,), jnp.float32)
    pltpu.sync_copy(zrow_v, acc_sh.at[pl.ds(sid * _RPS, _RPS)])
    plsc.subcore_barrier()
    for g in range(_NGRP):
        idesc[g].wait()
        for j in range(_RG):
            pltpu.sync_copy(ones_v, acc_sh.at[didx_v.at[g * _RG + j]],
                            add=True)
    plsc.subcore_barrier()
    pltpu.sync_copy(acc_sh.at[pl.ds(sid * _RPS, _RPS)],
                    out_hbm.at[pl.ds(cid * _NPAD + sid * _RPS, _RPS)])


@functools.partial(
    pl.kernel,
    out_type=jax.ShapeDtypeStruct((_NC * _NPAD, _H), jnp.float32),
    mesh=_sc_mesh,
    scratch_types=[
        pltpu.VMEM((max(_RA, _RB), _B), jnp.int32),
        pltpu.VMEM((max(_RA, _RB), _B), jnp.int32),
        pltpu.VMEM((3, _RGA, _B, _H), jnp.float32),
        pltpu.VMEM((64, _H), jnp.float32),
        pltpu.VMEM_SHARED((_NPAD, _H), jnp.float32),
        pltpu.VMEM_SHARED((_NPAD, _H), jnp.float32),
        pltpu.SemaphoreType.DMA,
        pltpu.SemaphoreType.DMA,
        pltpu.SemaphoreType.DMA,
        pltpu.SemaphoreType.DMA,
        pltpu.SemaphoreType.DMA,
        pltpu.SemaphoreType.DMA,
        pltpu.SemaphoreType.DMA,
        pltpu.SemaphoreType.DMA,
    ],
    compiler_params=pltpu.CompilerParams(use_tc_tiling_on_sc=False),
)
def _sc_agg(src_hbm, dst_hbm, t_hbm, out_hbm,
            sidx_v, didx_v, rows_v, zbuf_v, acc_sh, tab_sh,
            sem0, sem1, sem2, sem3, sem4, sem5, sem6, sem7):
    cid = lax.axis_index("c")
    sid = lax.axis_index("s")
    tdesc = pltpu.async_copy(t_hbm.at[pl.ds(sid * _RPS, _RPS)],
                             tab_sh.at[pl.ds(sid * _RPS, _RPS)], sem4)
    for r in range(64):
        for c in range(_H // 16):
            zbuf_v[r, pl.ds(c * 16, 16)] = jnp.zeros((16,), jnp.float32)
    for k in range(_RPS // 64):
        pltpu.sync_copy(zbuf_v, acc_sh.at[pl.ds(sid * _RPS + k * 64, 64)])
    tdesc.wait()
    plsc.subcore_barrier()
    gsems = (sem0, sem1, sem2)
    ssems = (sem3, sem6, sem7)

    def _edge_flow(row0, nrows):
        ngrp = nrows // _RGA
        idesc = []
        for g in range(ngrp):
            d1 = pltpu.async_copy(src_hbm.at[pl.ds(row0 + g * _RGA, _RGA)],
                                  sidx_v.at[pl.ds(g * _RGA, _RGA)], sem5)
            d2 = pltpu.async_copy(dst_hbm.at[pl.ds(row0 + g * _RGA, _RGA)],
                                  didx_v.at[pl.ds(g * _RGA, _RGA)], sem5)
            idesc.append((d1, d2))
        gpend = [None, None, None]
        spend = [None, None, None]

        def _fire(g):
            for d in idesc[g]:
                d.wait()
            buf = g % 3
            gpend[buf] = [
                pltpu.async_copy(tab_sh.at[sidx_v.at[g * _RGA + j]],
                                 rows_v.at[buf, j], gsems[buf])
                for j in range(_RGA)
            ]

        _fire(0)
        _fire(1)
        for g in range(ngrp):
            buf = g % 3
            if g >= 1:
                for d in spend[(g - 1) % 3]:
                    d.wait()
            if g + 2 < ngrp:
                _fire(g + 2)
            for d in gpend[buf]:
                d.wait()
            spend[buf] = [
                pltpu.async_copy(rows_v.at[buf, j],
                                 acc_sh.at[didx_v.at[g * _RGA + j]],
                                 ssems[buf], add=True)
                for j in range(_RGA)
            ]
        for d in spend[(ngrp - 1) % 3]:
            d.wait()

    @pl.when(cid == 0)
    def _core0():
        _edge_flow(sid * _RA, _RA)

    @pl.when(cid == 1)
    def _core1():
        _edge_flow(16 * _RA + sid * _RB, _RB)

    plsc.subcore_barrier()
    pltpu.sync_copy(acc_sh.at[pl.ds(sid * _RPS, _RPS)],
                    out_hbm.at[pl.ds(cid * _NPAD + sid * _RPS, _RPS)])


_P = _NPAD // 4
_BLKP = 640
_NBLKP = _P // _BLKP


def _tc1a_body(x4_ref, w_ref, out_ref):
    out_ref[...] = jnp.dot(x4_ref[...], w_ref[...],
                           preferred_element_type=jnp.float32)


_tc1a = pl.pallas_call(
    _tc1a_body,
    grid=(_NBLKP,),
    in_specs=[
        pl.BlockSpec((_BLKP, 4 * _F), lambda i: (i, 0)),
        pl.BlockSpec((4 * _F, 128), lambda i: (0, 0)),
    ],
    out_specs=pl.BlockSpec((_BLKP, 128), lambda i: (i, 0)),
    out_shape=jax.ShapeDtypeStruct((_P, 128), jnp.float32),
)


def _tc1b_body(degp0_ref, degp1_ref, xw_ref, t_ref, dinv_ref):
    dinv = lax.rsqrt(degp0_ref[...] + degp1_ref[...] + 1.0)
    t_ref[...] = xw_ref[...] * dinv
    dinv_ref[...] = dinv


_tc1b = pl.pallas_call(
    _tc1b_body,
    grid=(_NBLKP,),
    in_specs=[
        pl.BlockSpec((_BLKP, 128), lambda i: (i, 0)),
        pl.BlockSpec((_BLKP, 128), lambda i: (_NBLKP + i, 0)),
        pl.BlockSpec((_BLKP, 128), lambda i: (i, 0)),
    ],
    out_specs=[
        pl.BlockSpec((_BLKP, 128), lambda i: (i, 0)),
        pl.BlockSpec((_BLKP, 128), lambda i: (i, 0)),
    ],
    out_shape=[
        jax.ShapeDtypeStruct((_P, 128), jnp.float32),
        jax.ShapeDtypeStruct((_P, 128), jnp.float32),
    ],
)


def _tc_mid_body(sp0_ref, sp1_ref, t_ref, dinv_ref, b_ref, w_ref, out_ref):
    s = sp0_ref[...] + sp1_ref[...]
    dinv = dinv_ref[...]
    h = jnp.maximum((s + t_ref[...]) * dinv + b_ref[...], 0.0)
    out_ref[...] = jnp.dot(h, w_ref[...],
                           preferred_element_type=jnp.float32) * dinv


_tc_mid = pl.pallas_call(
    _tc_mid_body,
    grid=(_NBLKP,),
    in_specs=[
        pl.BlockSpec((_BLKP, 128), lambda i: (i, 0)),
        pl.BlockSpec((_BLKP, 128), lambda i: (_NBLKP + i, 0)),
        pl.BlockSpec((_BLKP, 128), lambda i: (i, 0)),
        pl.BlockSpec((_BLKP, 128), lambda i: (i, 0)),
        pl.BlockSpec((1, 128), lambda i: (0, 0)),
        pl.BlockSpec((128, 128), lambda i: (0, 0)),
    ],
    out_specs=pl.BlockSpec((_BLKP, 128), lambda i: (i, 0)),
    out_shape=jax.ShapeDtypeStruct((_P, 128), jnp.float32),
)


def _tc_final_body(sp0_ref, sp1_ref, t_ref, dinv_ref, b_ref, batch_ref,
                   wf1_ref, bf1_ref, wf2_ref, bf2_ref, out_ref,
                   acc_ref, cnt_ref):
    i = pl.program_id(0)

    @pl.when(i == 0)
    def _init():
        acc_ref[...] = jnp.zeros_like(acc_ref)
        cnt_ref[...] = jnp.zeros_like(cnt_ref)

    s = sp0_ref[...] + sp1_ref[...]
    dinv = dinv_ref[...]
    h = jnp.maximum((s + t_ref[...]) * dinv + b_ref[...], 0.0)
    gid = lax.broadcasted_iota(jnp.int32, (_G, _BLKP), 0)
    for a in range(4):
        mask = (gid == batch_ref[a, :][None, :]).astype(jnp.float32)
        acc_ref[...] += lax.dot_general(
            mask, h[:, 32 * a:32 * (a + 1)], (((1,), (0,)), ((), ())),
            preferred_element_type=jnp.float32)
        cnt_ref[...] += jnp.sum(mask, axis=1, keepdims=True)

    @pl.when(i == _NBLKP - 1)
    def _fin():
        pooled = acc_ref[...] / jnp.maximum(cnt_ref[...], 1.0)
        z = jnp.maximum(
            jnp.dot(pooled, wf1_ref[...],
                    preferred_element_type=jnp.float32) + bf1_ref[...], 0.0)
        z = jnp.dot(z, wf2_ref[...],
                    preferred_element_type=jnp.float32) + bf2_ref[...]
        m = jnp.max(z, axis=1, keepdims=True)
        lse = jnp.log(jnp.sum(jnp.exp(z - m), axis=1, keepdims=True)) + m
        out_ref[...] = z - lse


_tc_final = pl.pallas_call(
    _tc_final_body,
    grid=(_NBLKP,),
    in_specs=[
        pl.BlockSpec((_BLKP, 128), lambda i: (i, 0)),
        pl.BlockSpec((_BLKP, 128), lambda i: (_NBLKP + i, 0)),
        pl.BlockSpec((_BLKP, 128), lambda i: (i, 0)),
        pl.BlockSpec((_BLKP, 128), lambda i: (i, 0)),
        pl.BlockSpec((1, 128), lambda i: (0, 0)),
        pl.BlockSpec((4, _BLKP), lambda i: (0, i)),
        pl.BlockSpec((_H, _H), lambda i: (0, 0)),
        pl.BlockSpec((1, _H), lambda i: (0, 0)),
        pl.BlockSpec((_H, _C), lambda i: (0, 0)),
        pl.BlockSpec((1, _C), lambda i: (0, 0)),
    ],
    out_specs=pl.BlockSpec((_G, _C), lambda i: (0, 0)),
    out_shape=jax.ShapeDtypeStruct((_G, _C), jnp.float32),
    scratch_shapes=[
        pltpu.VMEM((_G, _H), jnp.float32),
        pltpu.VMEM((_G, 1), jnp.float32),
    ],
)


def _bdiag4(W):
    h, w = W.shape
    out = jnp.zeros((4 * h, 4 * w), jnp.float32)
    for a in range(4):
        out = out.at[a * h:(a + 1) * h, a * w:(a + 1) * w].set(W)
    return out


def kernel(x, edge_index, batch, W1, b1, W2, b2, W3, b3, W4, b4,
           Wf1, bf1, Wf2, bf2):
    src = edge_index[0]
    dst = edge_index[1]
    src_p = jnp.concatenate(
        [src, jnp.zeros((_EPAD - _E,), jnp.int32)]).reshape(_IDXROWS, _B)
    dst_p = jnp.concatenate(
        [dst, jnp.full((_EPAD - _E,), _NPAD - 1, jnp.int32)]).reshape(
            _IDXROWS, _B)
    x4 = jnp.pad(x, ((0, _NPAD - _N), (0, 0))).reshape(_P, 4 * _F)
    batch4 = jnp.pad(batch, (0, _NPAD - _N),
                     constant_values=_G).reshape(_P, 4).T
    w1bd = _bdiag4(W1)
    b14 = jnp.tile(b1, 4).reshape(1, 128)
    b24 = jnp.tile(b2, 4).reshape(1, 128)
    b34 = jnp.tile(b3, 4).reshape(1, 128)
    b44 = jnp.tile(b4, 4).reshape(1, 128)

    deg = _sc_degree(dst_p)
    xw = _tc1a(x4, w1bd)
    degp = jnp.repeat(deg, _H).reshape(2 * _P, 128)
    t1, dinvp = _tc1b(degp, degp, xw)

    def _lin(tp):
        return tp.reshape(_NPAD, _H)

    s = _sc_agg(src_p, dst_p, _lin(t1)).reshape(2 * _P, 128)
    t2 = _tc_mid(s, s, t1, dinvp, b14, _bdiag4(W2))
    s = _sc_agg(src_p, dst_p, _lin(t2)).reshape(2 * _P, 128)
    t3 = _tc_mid(s, s, t2, dinvp, b24, _bdiag4(W3))
    s = _sc_agg(src_p, dst_p, _lin(t3)).reshape(2 * _P, 128)
    t4 = _tc_mid(s, s, t3, dinvp, b34, _bdiag4(W4))
    s = _sc_agg(src_p, dst_p, _lin(t4)).reshape(2 * _P, 128)
    return _tc_final(s, s, t4, dinvp, b44, batch4,
                     Wf1, bf1.reshape(1, _H), Wf2, bf2.reshape(1, _C))


# speedup vs baseline: 1.0537x; 1.0537x over previous
import functools

import jax
import jax.numpy as jnp
from jax import lax
from jax.experimental import pallas as pl
from jax.experimental.pallas import tpu as pltpu
from jax.experimental.pallas import tpu_sc as plsc

_N = 10000
_E = 320000
_F = 128
_H = 32
_C = 10
_G = 64

_NC, _NS = 2, 16
_NW = _NC * _NS
_NPAD = 10240
_B = 128
_RG = 8
_EPAD = _NW * 80 * _B
_IDXROWS = _EPAD // _B
_ROWS_PER_W = _IDXROWS // _NW
_NGRP = _ROWS_PER_W // _RG
_RPS = _NPAD // _NS
_RGA = 4
_NGRPA = _ROWS_PER_W // _RGA
_RA = 88
_RB = 160 - _RA

_sc_mesh = plsc.VectorSubcoreMesh(core_axis_name="c", subcore_axis_name="s")


@functools.partial(
    pl.kernel,
    out_type=jax.ShapeDtypeStruct((_NC * _NPAD,), jnp.float32),
    mesh=_sc_mesh,
    scratch_types=[
        pltpu.VMEM((_ROWS_PER_W, _B), jnp.int32),
        pltpu.VMEM((_B,), jnp.float32),
        pltpu.VMEM((_RPS,), jnp.float32),
        pltpu.VMEM_SHARED((_NPAD,), jnp.float32),
        pltpu.SemaphoreType.DMA,
    ],
)
def _sc_degree(dst_hbm, out_hbm, didx_v, ones_v, zrow_v, acc_sh, sem):
    cid = lax.axis_index("c")
    sid = lax.axis_index("s")
    wid = sid * _NC + cid
    row0 = wid * _ROWS_PER_W
    idesc = [
        pltpu.async_copy(dst_hbm.at[pl.ds(row0 + g * _RG, _RG)],
                         didx_v.at[pl.ds(g * _RG, _RG)], sem)
        for g in range(_NGRP)
    ]
    for i in range(_B // 16):
        ones_v[pl.ds(i * 16, 16)] = jnp.ones((16,), jnp.float32)
    for i in range(_RPS // 16):
        zrow_v[pl.ds(i * 16, 16)] = jnp.zeros((16,), jnp.float32)
    pltpu.sync_copy(zrow_v, acc_sh.at[pl.ds(sid * _RPS, _RPS)])
    plsc.subcore_barrier()
    for g in range(_NGRP):
        idesc[g].wait()
        for j in range(_RG):
            pltpu.sync_copy(ones_v, acc_sh.at[didx_v.at[g * _RG + j]],
                            add=True)
    plsc.subcore_barrier()
    pltpu.sync_copy(acc_sh.at[pl.ds(sid * _RPS, _RPS)],
                    out_hbm.at[pl.ds(cid * _NPAD + sid * _RPS, _RPS)])


@functools.partial(
    pl.kernel,
    out_type=jax.ShapeDtypeStruct((_NC * _NPAD, _H), jnp.float32),
    mesh=_sc_mesh,
    scratch_types=[
        pltpu.VMEM((max(_RA, _RB), _B), jnp.int32),
        pltpu.VMEM((max(_RA, _RB), _B), jnp.int32),
        pltpu.VMEM((3, _RGA, _B, _H), jnp.float32),
        pltpu.VMEM((64, _H), jnp.float32),
        pltpu.VMEM_SHARED((_NPAD, _H), jnp.float32),
        pltpu.VMEM_SHARED((_NPAD, _H), jnp.float32),
        pltpu.SemaphoreType.DMA,
        pltpu.SemaphoreType.DMA,
        pltpu.SemaphoreType.DMA,
        pltpu.SemaphoreType.DMA,
        pltpu.SemaphoreType.DMA,
        pltpu.SemaphoreType.DMA,
        pltpu.SemaphoreType.DMA,
        pltpu.SemaphoreType.DMA,
    ],
    compiler_params=pltpu.CompilerParams(use_tc_tiling_on_sc=False),
)
def _sc_agg(src_hbm, dst_hbm, t_hbm, out_hbm,
            sidx_v, didx_v, rows_v, zbuf_v, acc_sh, tab_sh,
            sem0, sem1, sem2, sem3, sem4, sem5, sem6, sem7):
    cid = lax.axis_index("c")
    sid = lax.axis_index("s")
    tdesc = pltpu.async_copy(t_hbm.at[pl.ds(sid * _RPS, _RPS)],
                             tab_sh.at[pl.ds(sid * _RPS, _RPS)], sem4)
    for r in range(64):
        for c in range(_H // 16):
            zbuf_v[r, pl.ds(c * 16, 16)] = jnp.zeros((16,), jnp.float32)
    for k in range(_RPS // 64):
        pltpu.sync_copy(zbuf_v, acc_sh.at[pl.ds(sid * _RPS + k * 64, 64)])
    tdesc.wait()
    plsc.subcore_barrier()
    gsems = (sem0, sem1, sem2)
    ssems = (sem3, sem6, sem7)

    def _edge_flow(row0, nrows):
        ngrp = nrows // _RGA
        idesc = []
        for g in range(ngrp):
            d1 = pltpu.async_copy(src_hbm.at[pl.ds(row0 + g * _RGA, _RGA)],
                                  sidx_v.at[pl.ds(g * _RGA, _RGA)], sem5)
            d2 = pltpu.async_copy(dst_hbm.at[pl.ds(row0 + g * _RGA, _RGA)],
                                  didx_v.at[pl.ds(g * _RGA, _RGA)], sem5)
            idesc.append((d1, d2))
        gpend = [None, None, None]
        spend = [None, None, None]

        def _fire(g):
            for d in idesc[g]:
                d.wait()
            buf = g % 3
            gpend[buf] = [
                pltpu.async_copy(tab_sh.at[sidx_v.at[g * _RGA + j]],
                                 rows_v.at[buf, j], gsems[buf])
                for j in range(_RGA)
            ]

        _fire(0)
        _fire(1)
        for g in range(ngrp):
            buf = g % 3
            if g >= 1:
                for d in spend[(g - 1) % 3]:
                    d.wait()
            if g + 2 < ngrp:
                _fire(g + 2)
            for d in gpend[buf]:
                d.wait()
            spend[buf] = [
                pltpu.async_copy(rows_v.at[buf, j],
                                 acc_sh.at[didx_v.at[g * _RGA + j]],
                                 ssems[buf], add=True)
                for j in range(_RGA)
            ]
        for d in spend[(ngrp - 1) % 3]:
            d.wait()

    @pl.when(cid == 0)
    def _core0():
        _edge_flow(sid * _RA, _RA)

    @pl.when(cid == 1)
    def _core1():
        _edge_flow(16 * _RA + sid * _RB, _RB)

    plsc.subcore_barrier()
    pltpu.sync_copy(acc_sh.at[pl.ds(sid * _RPS, _RPS)],
                    out_hbm.at[pl.ds(cid * _NPAD + sid * _RPS, _RPS)])


_P = _NPAD // 4
_BLKP = 640
_NBLKP = _P // _BLKP


def _tc1a_body(x4_ref, w_ref, out_ref):
    out_ref[...] = jnp.dot(x4_ref[...], w_ref[...],
                           preferred_element_type=jnp.float32)


_tc1a = pl.pallas_call(
    _tc1a_body,
    grid=(_NBLKP,),
    in_specs=[
        pl.BlockSpec((_BLKP, 4 * _F), lambda i: (i, 0)),
        pl.BlockSpec((4 * _F, 128), lambda i: (0, 0)),
    ],
    out_specs=pl.BlockSpec((_BLKP, 128), lambda i: (i, 0)),
    out_shape=jax.ShapeDtypeStruct((_P, 128), jnp.float32),
)


def _tc1b_body(degp0_ref, degp1_ref, xw_ref, t_ref, dinv_ref):
    dinv = lax.rsqrt(degp0_ref[...] + degp1_ref[...] + 1.0)
    t_ref[...] = xw_ref[...] * dinv
    dinv_ref[...] = dinv


_tc1b = pl.pallas_call(
    _tc1b_body,
    grid=(_NBLKP,),
    in_specs=[
        pl.BlockSpec((_BLKP, 128), lambda i: (i, 0)),
        pl.BlockSpec((_BLKP, 128), lambda i: (_NBLKP + i, 0)),
        pl.BlockSpec((_BLKP, 128), lambda i: (i, 0)),
    ],
    out_specs=[
        pl.BlockSpec((_BLKP, 128), lambda i: (i, 0)),
        pl.BlockSpec((_BLKP, 128), lambda i: (i, 0)),
    ],
    out_shape=[
        jax.ShapeDtypeStruct((_P, 128), jnp.float32),
        jax.ShapeDtypeStruct((_P, 128), jnp.float32),
    ],
)


def _tc_mid_body(sp0_ref, sp1_ref, t_ref, dinv_ref, b_ref, w_ref, out_ref):
    s = sp0_ref[...] + sp1_ref[...]
    dinv = dinv_ref[...]
    h = jnp.maximum((s + t_ref[...]) * dinv + b_ref[...], 0.0)
    out_ref[...] = jnp.dot(h, w_ref[...],
                           preferred_element_type=jnp.float32) * dinv


_tc_mid = pl.pallas_call(
    _tc_mid_body,
    grid=(_NBLKP,),
    in_specs=[
        pl.BlockSpec((_BLKP, 128), lambda i: (i, 0)),
        pl.BlockSpec((_BLKP, 128), lambda i: (_NBLKP + i, 0)),
        pl.BlockSpec((_BLKP, 128), lambda i: (i, 0)),
        pl.BlockSpec((_BLKP, 128), lambda i: (i, 0)),
        pl.BlockSpec((1, 128), lambda i: (0, 0)),
        pl.BlockSpec((128, 128), lambda i: (0, 0)),
    ],
    out_specs=pl.BlockSpec((_BLKP, 128), lambda i: (i, 0)),
    out_shape=jax.ShapeDtypeStruct((_P, 128), jnp.float32),
)


def _tc_final_body(sp0_ref, sp1_ref, t_ref, dinv_ref, b_ref, batch_ref,
                   wf1_ref, bf1_ref, wf2_ref, bf2_ref, out_ref,
                   acc_ref, cnt_ref):
    i = pl.program_id(0)

    @pl.when(i == 0)
    def _init():
        acc_ref[...] = jnp.zeros_like(acc_ref)
        cnt_ref[...] = jnp.zeros_like(cnt_ref)

    s = sp0_ref[...] + sp1_ref[...]
    dinv = dinv_ref[...]
    h = jnp.maximum((s + t_ref[...]) * dinv + b_ref[...], 0.0)
    gid = lax.broadcasted_iota(jnp.int32, (_G, _BLKP), 0)
    for a in range(4):
        mask = (gid == batch_ref[a, :][None, :]).astype(jnp.float32)
        acc_ref[...] += lax.dot_general(
            mask, h[:, 32 * a:32 * (a + 1)], (((1,), (0,)), ((), ())),
            preferred_element_type=jnp.float32)
        cnt_ref[...] += jnp.sum(mask, axis=1, keepdims=True)

    @pl.when(i == _NBLKP - 1)
    def _fin():
        pooled = acc_ref[...] / jnp.maximum(cnt_ref[...], 1.0)
        z = jnp.maximum(
            jnp.dot(pooled, wf1_ref[...],
                    preferred_element_type=jnp.float32) + bf1_ref[...], 0.0)
        z = jnp.dot(z, wf2_ref[...],
                    preferred_element_type=jnp.float32) + bf2_ref[...]
        m = jnp.max(z, axis=1, keepdims=True)
        lse = jnp.log(jnp.sum(jnp.exp(z - m), axis=1, keepdims=True)) + m
        out_ref[...] = z - lse


_tc_final = pl.pallas_call(
    _tc_final_body,
    grid=(_NBLKP,),
    in_specs=[
        pl.BlockSpec((_BLKP, 128), lambda i: (i, 0)),
        pl.BlockSpec((_BLKP, 128), lambda i: (_NBLKP + i, 0)),
        pl.BlockSpec((_BLKP, 128), lambda i: (i, 0)),
        pl.BlockSpec((_BLKP, 128), lambda i: (i, 0)),
        pl.BlockSpec((1, 128), lambda i: (0, 0)),
        pl.BlockSpec((4, _BLKP), lambda i: (0, i)),
        pl.BlockSpec((_H, _H), lambda i: (0, 0)),
        pl.BlockSpec((1, _H), lambda i: (0, 0)),
        pl.BlockSpec((_H, _C), lambda i: (0, 0)),
        pl.BlockSpec((1, _C), lambda i: (0, 0)),
    ],
    out_specs=pl.BlockSpec((_G, _C), lambda i: (0, 0)),
    out_shape=jax.ShapeDtypeStruct((_G, _C), jnp.float32),
    scratch_shapes=[
        pltpu.VMEM((_G, _H), jnp.float32),
        pltpu.VMEM((_G, 1), jnp.float32),
    ],
)


def _bdiag4(W):
    h, w = W.shape
    out = jnp.zeros((4 * h, 4 * w), jnp.float32)
    for a in range(4):
        out = out.at[a * h:(a + 1) * h, a * w:(a + 1) * w].set(W)
    return out


def kernel(x, edge_index, batch, W1, b1, W2, b2, W3, b3, W4, b4,
           Wf1, bf1, Wf2, bf2):
    src = edge_index[0]
    dst = edge_index[1]
    src_p = jnp.concatenate(
        [src, jnp.zeros((_EPAD - _E,), jnp.int32)]).reshape(_IDXROWS, _B)
    dst_p = jnp.concatenate(
        [dst, jnp.full((_EPAD - _E,), _NPAD - 1, jnp.int32)]).reshape(
            _IDXROWS, _B)
    x4 = jnp.pad(x, ((0, _NPAD - _N), (0, 0))).reshape(_P, 4 * _F)
    batch4 = jnp.pad(batch, (0, _NPAD - _N),
                     constant_values=_G).reshape(_P, 4).T
    w1bd = _bdiag4(W1)
    b14 = jnp.tile(b1, 4).reshape(1, 128)
    b24 = jnp.tile(b2, 4).reshape(1, 128)
    b34 = jnp.tile(b3, 4).reshape(1, 128)
    b44 = jnp.tile(b4, 4).reshape(1, 128)

    deg = _sc_degree(dst_p)
    xw = _tc1a(x4, w1bd)
    degp = jnp.repeat(deg, _H).reshape(2 * _P, 128)
    t1, dinvp = _tc1b(degp, degp, xw)

    def _lin(tp):
        return tp.reshape(_NPAD, _H)

    s = _sc_agg(src_p, dst_p, _lin(t1)).reshape(2 * _P, 128)
    t2 = _tc_mid(s, s, t1, dinvp, b14, _bdiag4(W2))
    s = _sc_agg(src_p, dst_p, _lin(t2)).reshape(2 * _P, 128)
    t3 = _tc_mid(s, s, t2, dinvp, b24, _bdiag4(W3))
    s = _sc_agg(src_p, dst_p, _lin(t3)).reshape(2 * _P, 128)
    t4 = _tc_mid(s, s, t3, dinvp, b34, _bdiag4(W4))
    s = _sc_agg(src_p, dst_p, _lin(t4)).reshape(2 * _P, 128)
    return _tc_final(s, s, t4, dinvp, b44, batch4,
                     Wf1, bf1.reshape(1, _H), Wf2, bf2.reshape(1, _C))


# speedup vs baseline: 1.1339x; 1.0761x over previous
import functools

import jax
import jax.numpy as jnp
from jax import lax
from jax.experimental import pallas as pl
from jax.experimental.pallas import tpu as pltpu
from jax.experimental.pallas import tpu_sc as plsc

_N = 10000
_E = 320000
_F = 128
_H = 32
_C = 10
_G = 64

_NC, _NS = 2, 16
_NW = _NC * _NS
_NPAD = 10240
_B = 128
_RG = 8
_EPAD = _NW * 80 * _B
_IDXROWS = _EPAD // _B
_ROWS_PER_W = _IDXROWS // _NW
_NGRP = _ROWS_PER_W // _RG
_RPS = _NPAD // _NS
_RGA = 4
_NGRPA = _ROWS_PER_W // _RGA
_RA = 80
_RB = 160 - _RA

_sc_mesh = plsc.VectorSubcoreMesh(core_axis_name="c", subcore_axis_name="s")


@functools.partial(
    pl.kernel,
    out_type=jax.ShapeDtypeStruct((_NC * _NPAD,), jnp.float32),
    mesh=_sc_mesh,
    scratch_types=[
        pltpu.VMEM((_ROWS_PER_W, _B), jnp.int32),
        pltpu.VMEM((_B,), jnp.float32),
        pltpu.VMEM((_RPS,), jnp.float32),
        pltpu.VMEM_SHARED((_NPAD,), jnp.float32),
        pltpu.SemaphoreType.DMA,
    ],
)
def _sc_degree(e_hbm, out_hbm, didx_v, ones_v, zrow_v, acc_sh, sem):
    cid = lax.axis_index("c")
    sid = lax.axis_index("s")
    wid = sid * _NC + cid
    row0 = wid * _ROWS_PER_W
    idesc = [
        pltpu.async_copy(e_hbm.at[1, pl.ds(row0 + g * _RG, _RG)],
                         didx_v.at[pl.ds(g * _RG, _RG)], sem)
        for g in range(_NGRP)
    ]
    for i in range(_B // 16):
        ones_v[pl.ds(i * 16, 16)] = jnp.ones((16,), jnp.float32)
    for i in range(_RPS // 16):
        zrow_v[pl.ds(i * 16, 16)] = jnp.zeros((16,), jnp.float32)
    pltpu.sync_copy(zrow_v, acc_sh.at[pl.ds(sid * _RPS, _RPS)])
    plsc.subcore_barrier()
    for g in range(_NGRP):
        idesc[g].wait()
        for j in range(_RG):
            pltpu.sync_copy(ones_v, acc_sh.at[didx_v.at[g * _RG + j]],
                            add=True)
    plsc.subcore_barrier()
    pltpu.sync_copy(acc_sh.at[pl.ds(sid * _RPS, _RPS)],
                    out_hbm.at[pl.ds(cid * _NPAD + sid * _RPS, _RPS)])


@functools.partial(
    pl.kernel,
    out_type=jax.ShapeDtypeStruct((_NC * _NPAD, _H), jnp.float32),
    mesh=_sc_mesh,
    scratch_types=[
        pltpu.VMEM((max(_RA, _RB), _B), jnp.int32),
        pltpu.VMEM((max(_RA, _RB), _B), jnp.int32),
        pltpu.VMEM((3, _RGA, _B, _H), jnp.float32),
        pltpu.VMEM((64, _H), jnp.float32),
        pltpu.VMEM_SHARED((_NPAD, _H), jnp.float32),
        pltpu.VMEM_SHARED((_NPAD, _H), jnp.float32),
        pltpu.SemaphoreType.DMA,
        pltpu.SemaphoreType.DMA,
        pltpu.SemaphoreType.DMA,
        pltpu.SemaphoreType.DMA,
        pltpu.SemaphoreType.DMA,
        pltpu.SemaphoreType.DMA,
        pltpu.SemaphoreType.DMA,
        pltpu.SemaphoreType.DMA,
    ],
    compiler_params=pltpu.CompilerParams(use_tc_tiling_on_sc=False),
)
def _sc_agg(e_hbm, t_hbm, out_hbm,
            sidx_v, didx_v, rows_v, zbuf_v, acc_sh, tab_sh,
            sem0, sem1, sem2, sem3, sem4, sem5, sem6, sem7):
    cid = lax.axis_index("c")
    sid = lax.axis_index("s")
    tdesc = pltpu.async_copy(t_hbm.at[pl.ds(sid * _RPS, _RPS)],
                             tab_sh.at[pl.ds(sid * _RPS, _RPS)], sem4)
    for r in range(64):
        for c in range(_H // 16):
            zbuf_v[r, pl.ds(c * 16, 16)] = jnp.zeros((16,), jnp.float32)
    for k in range(_RPS // 64):
        pltpu.sync_copy(zbuf_v, acc_sh.at[pl.ds(sid * _RPS + k * 64, 64)])
    tdesc.wait()
    plsc.subcore_barrier()
    gsems = (sem0, sem1, sem2)
    ssems = (sem3, sem6, sem7)

    def _edge_flow(row0, nrows):
        ngrp = nrows // _RGA
        idesc = []
        for g in range(ngrp):
            d1 = pltpu.async_copy(e_hbm.at[0, pl.ds(row0 + g * _RGA, _RGA)],
                                  sidx_v.at[pl.ds(g * _RGA, _RGA)], sem5)
            d2 = pltpu.async_copy(e_hbm.at[1, pl.ds(row0 + g * _RGA, _RGA)],
                                  didx_v.at[pl.ds(g * _RGA, _RGA)], sem5)
            idesc.append((d1, d2))
        gpend = [None, None, None]
        spend = [None, None, None]

        def _fire(g):
            for d in idesc[g]:
                d.wait()
            buf = g % 3
            gpend[buf] = [
                pltpu.async_copy(tab_sh.at[sidx_v.at[g * _RGA + j]],
                                 rows_v.at[buf, j], gsems[buf])
                for j in range(_RGA)
            ]

        _fire(0)
        _fire(1)
        for g in range(ngrp):
            buf = g % 3
            if g >= 1:
                for d in spend[(g - 1) % 3]:
                    d.wait()
            if g + 2 < ngrp:
                _fire(g + 2)
            for d in gpend[buf]:
                d.wait()
            spend[buf] = [
                pltpu.async_copy(rows_v.at[buf, j],
                                 acc_sh.at[didx_v.at[g * _RGA + j]],
                                 ssems[buf], add=True)
                for j in range(_RGA)
            ]
        for d in spend[(ngrp - 1) % 3]:
            d.wait()

    @pl.when(cid == 0)
    def _core0():
        _edge_flow(sid * _RA, _RA)

    @pl.when(cid == 1)
    def _core1():
        _edge_flow(16 * _RA + sid * _RB, _RB)

    plsc.subcore_barrier()
    pltpu.sync_copy(acc_sh.at[pl.ds(sid * _RPS, _RPS)],
                    out_hbm.at[pl.ds(cid * _NPAD + sid * _RPS, _RPS)])


_P = _NPAD // 4
_BLKP = 640
_NBLKP = _P // _BLKP


def _tc1a_body(x4_ref, w_ref, out_ref):
    out_ref[...] = jnp.dot(x4_ref[...], w_ref[...],
                           preferred_element_type=jnp.float32)


_tc1a = pl.pallas_call(
    _tc1a_body,
    grid=(_NBLKP,),
    in_specs=[
        pl.BlockSpec((_BLKP, 4 * _F), lambda i: (i, 0)),
        pl.BlockSpec((4 * _F, 128), lambda i: (0, 0)),
    ],
    out_specs=pl.BlockSpec((_BLKP, 128), lambda i: (i, 0)),
    out_shape=jax.ShapeDtypeStruct((_P, 128), jnp.float32),
)


def _tc1b_body(degp0_ref, degp1_ref, xw_ref, t_ref, dinv_ref):
    dinv = lax.rsqrt(degp0_ref[...] + degp1_ref[...] + 1.0)
    t_ref[...] = xw_ref[...] * dinv
    dinv_ref[...] = dinv


_tc1b = pl.pallas_call(
    _tc1b_body,
    grid=(_NBLKP,),
    in_specs=[
        pl.BlockSpec((_BLKP, 128), lambda i: (i, 0)),
        pl.BlockSpec((_BLKP, 128), lambda i: (_NBLKP + i, 0)),
        pl.BlockSpec((_BLKP, 128), lambda i: (i, 0)),
    ],
    out_specs=[
        pl.BlockSpec((_BLKP, 128), lambda i: (i, 0)),
        pl.BlockSpec((_BLKP, 128), lambda i: (i, 0)),
    ],
    out_shape=[
        jax.ShapeDtypeStruct((_P, 128), jnp.float32),
        jax.ShapeDtypeStruct((_P, 128), jnp.float32),
    ],
)


def _tc_mid_body(sp0_ref, sp1_ref, t_ref, dinv_ref, b_ref, w_ref, out_ref):
    s = sp0_ref[...] + sp1_ref[...]
    dinv = dinv_ref[...]
    h = jnp.maximum((s + t_ref[...]) * dinv + b_ref[...], 0.0)
    out_ref[...] = jnp.dot(h, w_ref[...],
                           preferred_element_type=jnp.float32) * dinv


_tc_mid = pl.pallas_call(
    _tc_mid_body,
    grid=(_NBLKP,),
    in_specs=[
        pl.BlockSpec((_BLKP, 128), lambda i: (i, 0)),
        pl.BlockSpec((_BLKP, 128), lambda i: (_NBLKP + i, 0)),
        pl.BlockSpec((_BLKP, 128), lambda i: (i, 0)),
        pl.BlockSpec((_BLKP, 128), lambda i: (i, 0)),
        pl.BlockSpec((1, 128), lambda i: (0, 0)),
        pl.BlockSpec((128, 128), lambda i: (0, 0)),
    ],
    out_specs=pl.BlockSpec((_BLKP, 128), lambda i: (i, 0)),
    out_shape=jax.ShapeDtypeStruct((_P, 128), jnp.float32),
)


def _tc_final_body(sp0_ref, sp1_ref, t_ref, dinv_ref, b_ref, batch_ref,
                   wf1_ref, bf1_ref, wf2_ref, bf2_ref, out_ref,
                   acc_ref, cnt_ref):
    i = pl.program_id(0)

    @pl.when(i == 0)
    def _init():
        acc_ref[...] = jnp.zeros_like(acc_ref)
        cnt_ref[...] = jnp.zeros_like(cnt_ref)

    s = sp0_ref[...] + sp1_ref[...]
    dinv = dinv_ref[...]
    h = jnp.maximum((s + t_ref[...]) * dinv + b_ref[...], 0.0)
    gid = lax.broadcasted_iota(jnp.int32, (_G, _BLKP), 0)
    for a in range(4):
        mask = (gid == batch_ref[a, :][None, :]).astype(jnp.float32)
        acc_ref[...] += lax.dot_general(
            mask, h[:, 32 * a:32 * (a + 1)], (((1,), (0,)), ((), ())),
            preferred_element_type=jnp.float32)
        cnt_ref[...] += jnp.sum(mask, axis=1, keepdims=True)

    @pl.when(i == _NBLKP - 1)
    def _fin():
        pooled = acc_ref[...] / jnp.maximum(cnt_ref[...], 1.0)
        z = jnp.maximum(
            jnp.dot(pooled, wf1_ref[...],
                    preferred_element_type=jnp.float32) + bf1_ref[...], 0.0)
        z = jnp.dot(z, wf2_ref[...],
                    preferred_element_type=jnp.float32) + bf2_ref[...]
        m = jnp.max(z, axis=1, keepdims=True)
        lse = jnp.log(jnp.sum(jnp.exp(z - m), axis=1, keepdims=True)) + m
        out_ref[...] = z - lse


_tc_final = pl.pallas_call(
    _tc_final_body,
    grid=(_NBLKP,),
    in_specs=[
        pl.BlockSpec((_BLKP, 128), lambda i: (i, 0)),
        pl.BlockSpec((_BLKP, 128), lambda i: (_NBLKP + i, 0)),
        pl.BlockSpec((_BLKP, 128), lambda i: (i, 0)),
        pl.BlockSpec((_BLKP, 128), lambda i: (i, 0)),
        pl.BlockSpec((1, 128), lambda i: (0, 0)),
        pl.BlockSpec((4, _BLKP), lambda i: (0, i)),
        pl.BlockSpec((_H, _H), lambda i: (0, 0)),
        pl.BlockSpec((1, _H), lambda i: (0, 0)),
        pl.BlockSpec((_H, _C), lambda i: (0, 0)),
        pl.BlockSpec((1, _C), lambda i: (0, 0)),
    ],
    out_specs=pl.BlockSpec((_G, _C), lambda i: (0, 0)),
    out_shape=jax.ShapeDtypeStruct((_G, _C), jnp.float32),
    scratch_shapes=[
        pltpu.VMEM((_G, _H), jnp.float32),
        pltpu.VMEM((_G, 1), jnp.float32),
    ],
)


def _bdiag4(W):
    h, w = W.shape
    out = jnp.zeros((4 * h, 4 * w), jnp.float32)
    for a in range(4):
        out = out.at[a * h:(a + 1) * h, a * w:(a + 1) * w].set(W)
    return out


def kernel(x, edge_index, batch, W1, b1, W2, b2, W3, b3, W4, b4,
           Wf1, bf1, Wf2, bf2):
    ei_p = jnp.pad(edge_index, ((0, 0), (0, _EPAD - _E)),
                   constant_values=_N).reshape(2, _IDXROWS, _B)
    x4 = jnp.pad(x, ((0, _NPAD - _N), (0, 0))).reshape(_P, 4 * _F)
    batch4 = jnp.pad(batch, (0, _NPAD - _N),
                     constant_values=_G).reshape(_P, 4).T
    w1bd = _bdiag4(W1)
    b14 = jnp.tile(b1, 4).reshape(1, 128)
    b24 = jnp.tile(b2, 4).reshape(1, 128)
    b34 = jnp.tile(b3, 4).reshape(1, 128)
    b44 = jnp.tile(b4, 4).reshape(1, 128)

    deg = _sc_degree(ei_p)
    xw = _tc1a(x4, w1bd)
    degp = jnp.repeat(deg, _H).reshape(2 * _P, 128)
    t1, dinvp = _tc1b(degp, degp, xw)

    def _lin(tp):
        return tp.reshape(_NPAD, _H)

    s = _sc_agg(ei_p, _lin(t1)).reshape(2 * _P, 128)
    t2 = _tc_mid(s, s, t1, dinvp, b14, _bdiag4(W2))
    s = _sc_agg(ei_p, _lin(t2)).reshape(2 * _P, 128)
    t3 = _tc_mid(s, s, t2, dinvp, b24, _bdiag4(W3))
    s = _sc_agg(ei_p, _lin(t3)).reshape(2 * _P, 128)
    t4 = _tc_mid(s, s, t3, dinvp, b34, _bdiag4(W4))
    s = _sc_agg(ei_p, _lin(t4)).reshape(2 * _P, 128)
    return _tc_final(s, s, t4, dinvp, b44, batch4,
                     Wf1, bf1.reshape(1, _H), Wf2, bf2.reshape(1, _C))
